# baseline (speedup 1.0000x reference)
import functools

import jax
import jax.numpy as jnp
from jax import lax
from jax.experimental import pallas as pl
from jax.experimental.pallas import tpu as pltpu
from jax.experimental.pallas import tpu_sc as plsc

VOCAB = 1000000
DIM = 64
BATCH = 16384

NC = 2
NS = 16
NW = NC * NS
TROW = 8
CW = 128
NFULL = (VOCAB - DIM) // CW
NCW = 245
CLPAD = NCW
NRING = 8
NGRP = BATCH // 16
WLCAP = BATCH + 16

_mesh = plsc.VectorSubcoreMesh(core_axis_name="c", subcore_axis_name="s")


@functools.partial(
    pl.kernel,
    mesh=_mesh,
    compiler_params=pltpu.CompilerParams(
        use_tc_tiling_on_sc=True, needs_layout_passes=False),
    out_type=jax.ShapeDtypeStruct((BATCH // TROW, TROW, DIM), jnp.float32),
    scratch_types=[
        pltpu.VMEM((BATCH,), jnp.int32),
        pltpu.VMEM((WLCAP,), jnp.int32),
        pltpu.VMEM((WLCAP,), jnp.int32),
        pltpu.VMEM((DIM, CW), jnp.float32),
        pltpu.VMEM((DIM, CW), jnp.float32),
        pltpu.VMEM((DIM, CW), jnp.float32),
        pltpu.VMEM((DIM, CW), jnp.float32),
        pltpu.VMEM((DIM, CW), jnp.float32),
        pltpu.VMEM((DIM, CW), jnp.float32),
        pltpu.VMEM((DIM, CW), jnp.float32),
        pltpu.VMEM((DIM, CW), jnp.float32),
        pltpu.VMEM((2, TROW, DIM), jnp.float32),
        pltpu.SMEM((NCW + 3,), jnp.int32),
        pltpu.SMEM((NCW + 3,), jnp.int32),
        pltpu.SMEM((NCW + 24,), jnp.int32),
        pltpu.SemaphoreType.DMA,
        pltpu.SemaphoreType.DMA,
        pltpu.SemaphoreType.DMA,
        pltpu.SemaphoreType.DMA,
        pltpu.SemaphoreType.DMA,
        pltpu.SemaphoreType.DMA,
        pltpu.SemaphoreType.DMA,
        pltpu.SemaphoreType.DMA,
        pltpu.SemaphoreType.DMA,
    ],
)
def _scan_gather(idx_hbm, wt_hbm, tail_hbm, out_hbm, idx_all, wl, swl,
                 buf0, buf1, buf2, buf3, buf4, buf5, buf6, buf7, rowb,
                 offs, cur, nel, gsem0, gsem1, gsem2, gsem3, gsem4, gsem5,
                 gsem6, gsem7, wsem):
    wid = lax.axis_index("s") * NC + lax.axis_index("c")
    iota16 = lax.iota(jnp.int32, 16)
    lane0 = iota16 == 0
    bufs = (buf0, buf1, buf2, buf3, buf4, buf5, buf6, buf7)
    gsems = (gsem0, gsem1, gsem2, gsem3, gsem4, gsem5, gsem6, gsem7)

    pltpu.sync_copy(idx_hbm, idx_all)

    def p1_body(g, n):
        v = idx_all[pl.ds(g * 16, 16)]
        chunk = lax.shift_right_logical(v, 7)
        cl = lax.shift_right_logical(chunk, 5)
        m = lax.bitwise_and(chunk, 31) == wid
        b = iota16 + g * 16
        entry = (lax.shift_left(cl, 21)
                 | lax.shift_left(b, 7)
                 | lax.bitwise_and(v, CW - 1))
        plsc.store_compressed(wl.at[pl.ds(n, 16)], entry, mask=m)
        cnt = plsc.all_reduce_population_count(m)
        return n + cnt[0]

    n = lax.fori_loop(0, NGRP, p1_body, jnp.int32(0))

    def zero_body(s, c):
        offs[s] = jnp.int32(0)
        cur[s] = jnp.int32(0)
        return c

    lax.fori_loop(0, NCW + 3, zero_body, jnp.int32(0))

    def count_body(j, c):
        e = wl[pl.ds(j, 16)][0]
        cl = lax.shift_right_logical(e, 21)
        cur[cl] = cur[cl] + 1
        return c

    lax.fori_loop(0, n, count_body, jnp.int32(0))

    def ne_body(s, ne):
        keep = cur[s] > 0

        @pl.when(keep)
        def _():
            nel[ne] = s

        return ne + keep.astype(jnp.int32)

    ne = lax.fori_loop(0, NCW, ne_body, jnp.int32(0))

    def nepad_body(k, c):
        nel[ne + k] = jnp.int32(CLPAD)
        return c

    lax.fori_loop(0, 2 * NRING, nepad_body, jnp.int32(0))

    def prefix_body(s, acc):
        offs[s] = acc
        acc = acc + cur[s]
        cur[s] = offs[s]
        return acc

    total = lax.fori_loop(0, NCW, prefix_body, jnp.int32(0))
    offs[NCW] = total
    offs[NCW + 1] = total

    def place_body(j, c):
        e = wl[pl.ds(j, 16)][0]
        cl = lax.shift_right_logical(e, 21)
        d = cur[cl]
        cur[cl] = d + 1
        plsc.store_scatter(swl, [jnp.full((16,), d, jnp.int32)],
                           jnp.full((16,), e, jnp.int32), mask=lane0)
        return c

    lax.fori_loop(0, n, place_body, jnp.int32(0))

    def fire_q(q, r):
        ch = lax.shift_left(nel[q], 5) + wid

        @pl.when(ch < NFULL)
        def _():
            pltpu.async_copy(
                wt_hbm.at[:, pl.ds(pl.multiple_of(ch * CW, CW), CW)],
                bufs[r], gsems[r])

        @pl.when(ch >= NFULL)
        def _():
            pltpu.async_copy(tail_hbm, bufs[r], gsems[r])

    def drain_q(r):
        pltpu.make_async_copy(wt_hbm.at[:, pl.ds(0, CW)], bufs[r],
                              gsems[r]).wait()

    def extract(j, buf):
        e = swl[pl.ds(j, 16)][0]
        b = lax.bitwise_and(lax.shift_right_logical(e, 7),
                            jnp.int32(BATCH - 1))
        csplat = jnp.full((16,), lax.bitwise_and(e, CW - 1), jnp.int32)
        slot = lax.bitwise_and(j, 15)
        t = lax.shift_right_logical(slot, 3)
        s = lax.bitwise_and(slot, 7)

        @pl.when(j >= 16)
        def _():
            pltpu.make_async_copy(rowb.at[0, 0], out_hbm.at[0, 0],
                                  wsem).wait()

        for g in range(DIM // 16):
            vals = plsc.load_gather(buf, [iota16 + g * 16, csplat])
            plsc.store_scatter(
                rowb, [jnp.full((16,), t, jnp.int32),
                       jnp.full((16,), s, jnp.int32), iota16 + g * 16],
                vals)
        pltpu.async_copy(
            rowb.at[t, s],
            out_hbm.at[lax.shift_right_logical(b, 3),
                       lax.bitwise_and(b, 7)],
            wsem)

    for r in range(NRING):
        fire_q(jnp.int32(r), r)

    def p3_body(q8, c):
        q0 = q8 * NRING
        for r in range(NRING):
            q = q0 + r
            drain_q(r)
            cl_q = nel[q]

            def ext_body(j, cc):
                extract(j, bufs[r])
                return cc

            lax.fori_loop(offs[cl_q], offs[cl_q + 1], ext_body,
                          jnp.int32(0))
            fire_q(q + NRING, r)
        return c

    nsweep = lax.shift_right_logical(ne + NRING - 1, 3)
    lax.fori_loop(0, nsweep, p3_body, jnp.int32(0))

    for r in range(NRING):
        drain_q(r)

    def wdrain_body(k, c):
        pltpu.make_async_copy(rowb.at[0, 0], out_hbm.at[0, 0], wsem).wait()
        return c

    lax.fori_loop(0, jnp.minimum(n, 16), wdrain_body, jnp.int32(0))


def kernel(inputs, W):
    wt = W.T
    tail = jnp.pad(wt[:, VOCAB - DIM:], ((0, 0), (0, CW - DIM)))
    out3 = _scan_gather(inputs, wt, tail)
    return out3.reshape(BATCH, DIM)


# speedup vs baseline: 1.0460x; 1.0460x over previous
import functools

import jax
import jax.numpy as jnp
from jax import lax
from jax.experimental import pallas as pl
from jax.experimental.pallas import tpu as pltpu
from jax.experimental.pallas import tpu_sc as plsc

VOCAB = 1000000
DIM = 64
BATCH = 16384

NC = 2
NS = 16
NW = NC * NS
TROW = 8
CW = 256
NCHUNK = (VOCAB + CW - 1) // CW
NCHUNK_W = 124
NRING = 4
NGRP = BATCH // 16
WLCAP = BATCH + 16

_mesh = plsc.VectorSubcoreMesh(core_axis_name="c", subcore_axis_name="s")


@functools.partial(
    pl.kernel,
    mesh=_mesh,
    compiler_params=pltpu.CompilerParams(
        use_tc_tiling_on_sc=True, needs_layout_passes=False),
    out_type=jax.ShapeDtypeStruct((BATCH // TROW, TROW, DIM), jnp.float32),
    scratch_types=[
        pltpu.VMEM((BATCH,), jnp.int32),
        pltpu.VMEM((WLCAP,), jnp.int32),
        pltpu.VMEM((WLCAP,), jnp.int32),
        pltpu.VMEM((DIM, CW), jnp.float32),
        pltpu.VMEM((DIM, CW), jnp.float32),
        pltpu.VMEM((DIM, CW), jnp.float32),
        pltpu.VMEM((DIM, CW), jnp.float32),
        pltpu.VMEM((2, TROW, DIM), jnp.float32),
        pltpu.SMEM((NCHUNK_W + 1,), jnp.int32),
        pltpu.SMEM((NCHUNK_W + 1,), jnp.int32),
        pltpu.SemaphoreType.DMA,
        pltpu.SemaphoreType.DMA,
        pltpu.SemaphoreType.DMA,
        pltpu.SemaphoreType.DMA,
        pltpu.SemaphoreType.DMA,
    ],
)
def _scan_gather(idx_hbm, wt_hbm, tail_hbm, out_hbm, idx_all, wl, swl,
                 buf0, buf1, buf2, buf3, rowb, offs, cur, gsem0, gsem1,
                 gsem2, gsem3, wsem):
    wid = lax.axis_index("s") * NC + lax.axis_index("c")
    iota16 = lax.iota(jnp.int32, 16)
    lane0 = iota16 == 0

    bufs = (buf0, buf1, buf2, buf3)
    gsems = (gsem0, gsem1, gsem2, gsem3)
    NFULL = (VOCAB - DIM) // CW

    def fire_chunk(i, r):
        ch = wid + lax.shift_left(i, 5)

        @pl.when(ch < NFULL)
        def _():
            pltpu.async_copy(
                wt_hbm.at[:, pl.ds(pl.multiple_of(ch * CW, CW), CW)],
                bufs[r], gsems[r])

        @pl.when(ch >= NFULL)
        def _():
            pltpu.async_copy(tail_hbm, bufs[r], gsems[r])

    for r in range(NRING):
        fire_chunk(jnp.int32(r), r)

    pltpu.sync_copy(idx_hbm, idx_all)

    def p1_body(g, n):
        v = idx_all[pl.ds(g * 16, 16)]
        chunk = lax.shift_right_logical(v, 8)
        cl = lax.shift_right_logical(chunk, 5)
        m = lax.bitwise_and(chunk, 31) == wid
        b = iota16 + g * 16
        entry = (lax.shift_left(cl, 22)
                 | lax.shift_left(b, 8)
                 | lax.bitwise_and(v, CW - 1))
        plsc.store_compressed(wl.at[pl.ds(n, 16)], entry, mask=m)
        cnt = plsc.all_reduce_population_count(m)
        return n + cnt[0]

    n = lax.fori_loop(0, NGRP, p1_body, jnp.int32(0))

    def zero_body(s, c):
        offs[s] = jnp.int32(0)
        cur[s] = jnp.int32(0)
        return c

    lax.fori_loop(0, NCHUNK_W + 1, zero_body, jnp.int32(0))

    def count_body(j, c):
        e = wl[pl.ds(j, 16)][0]
        cl = lax.shift_right_logical(e, 22)
        cur[cl] = cur[cl] + 1
        return c

    lax.fori_loop(0, n, count_body, jnp.int32(0))

    def prefix_body(s, acc):
        offs[s] = acc
        acc = acc + cur[s]
        cur[s] = offs[s]
        return acc

    total = lax.fori_loop(0, NCHUNK_W, prefix_body, jnp.int32(0))
    offs[NCHUNK_W] = total

    def place_body(j, c):
        e = wl[pl.ds(j, 16)][0]
        cl = lax.shift_right_logical(e, 22)
        d = cur[cl]
        cur[cl] = d + 1
        plsc.store_scatter(swl, [jnp.full((16,), d, jnp.int32)],
                           jnp.full((16,), e, jnp.int32), mask=lane0)
        return c

    lax.fori_loop(0, n, place_body, jnp.int32(0))


    def drain_chunk(r):
        pltpu.make_async_copy(wt_hbm.at[:, pl.ds(0, CW)], bufs[r],
                              gsems[r]).wait()

    def extract(j, buf):
        e = swl[pl.ds(j, 16)][0]
        b = lax.bitwise_and(lax.shift_right_logical(e, 8),
                            jnp.int32(BATCH - 1))
        csplat = jnp.full((16,), lax.bitwise_and(e, CW - 1), jnp.int32)
        slot = lax.bitwise_and(j, 15)
        t = lax.shift_right_logical(slot, 3)
        s = lax.bitwise_and(slot, 7)

        @pl.when(j >= 16)
        def _():
            pltpu.make_async_copy(rowb.at[0, 0], out_hbm.at[0, 0],
                                  wsem).wait()

        for g in range(DIM // 16):
            vals = plsc.load_gather(buf, [iota16 + g * 16, csplat])
            plsc.store_scatter(
                rowb, [jnp.full((16,), t, jnp.int32),
                       jnp.full((16,), s, jnp.int32), iota16 + g * 16],
                vals)
        pltpu.async_copy(
            rowb.at[t, s],
            out_hbm.at[lax.shift_right_logical(b, 3),
                       lax.bitwise_and(b, 7)],
            wsem)

    def p3_body(q, c):
        i0 = q * NRING
        for r in range(NRING):
            i = i0 + r
            drain_chunk(r)

            def ext_body(j, cc):
                extract(j, bufs[r])
                return cc

            lax.fori_loop(offs[i], offs[i + 1], ext_body, jnp.int32(0))
            fire_chunk(i + NRING, r)
        return c

    lax.fori_loop(0, NCHUNK_W // NRING - 1, p3_body, jnp.int32(0))

    def p3_last(r):
        i = NCHUNK_W - NRING + r
        drain_chunk(r)

        def ext_body(j, cc):
            extract(j, bufs[r])
            return cc

        lax.fori_loop(offs[i], offs[i + 1], ext_body, jnp.int32(0))

    for r in range(NRING):
        p3_last(r)

    def wdrain_body(k, c):
        pltpu.make_async_copy(rowb.at[0, 0], out_hbm.at[0, 0], wsem).wait()
        return c

    lax.fori_loop(0, jnp.minimum(n, 16), wdrain_body, jnp.int32(0))


def kernel(inputs, W):
    wt = W.T
    tail = jnp.pad(wt[:, VOCAB - DIM:], ((0, 0), (0, CW - DIM)))
    out3 = _scan_gather(inputs, wt, tail)
    return out3.reshape(BATCH, DIM)
